# Initial kernel scaffold; baseline (speedup 1.0000x reference)
import jax
import jax.numpy as jnp
from jax.experimental import pallas as pl


def kernel(x, batch_indices):
    raise NotImplementedError("write your pallas kernel here")


# speedup vs baseline: 4.0243x; 4.0243x over previous
import functools

import jax
import jax.numpy as jnp
from jax import lax
from jax.experimental import pallas as pl
from jax.experimental.pallas import tpu as pltpu
from jax.experimental.pallas import tpu_sc as plsc

N = 100000
D = 128
S = 1024
NC = 2
NS = 16
NW = NC * NS
BASE = 3128
CH = 128
ACC_ROWS = 1040
DUMMY = 1024


def _sc_body(x_hbm, ids_hbm, out_hbm, acc, zbuf, rows_buf, ids_buf):
    c = lax.axis_index("c")
    s = lax.axis_index("s")
    wid = c * NS + s
    row0 = wid * BASE
    rows_w = jnp.minimum(BASE, N - row0)
    full = rows_w // CH
    tail8 = (rows_w - full * CH) // 8

    zeros16 = jnp.zeros((16,), jnp.float32)

    def zrow(i, carry):
        for k in range(D // 16):
            zbuf[i, pl.ds(k * 16, 16)] = zeros16
        return carry

    lax.fori_loop(0, ACC_ROWS // NS, zrow, 0)
    pltpu.sync_copy(zbuf, acc.at[pl.ds(s * (ACC_ROWS // NS), ACC_ROWS // NS)])
    plsc.subcore_barrier()

    def chunk(j, carry):
        r0 = row0 + j * CH
        pltpu.sync_copy(ids_hbm.at[pl.ds(r0, CH)], ids_buf)
        pltpu.sync_copy(x_hbm.at[pl.ds(r0, CH), :], rows_buf)
        pltpu.sync_copy(rows_buf, acc.at[ids_buf], add=True)
        return carry

    lax.fori_loop(0, full, chunk, 0)

    dummy16 = jnp.full((16,), DUMMY, jnp.int32)
    for k in range(CH // 16):
        ids_buf[pl.ds(k * 16, 16)] = dummy16
    t0 = row0 + full * CH

    def tail(t, carry):
        r = t0 + t * 8
        pltpu.sync_copy(ids_hbm.at[pl.ds(r, 8)], ids_buf.at[pl.ds(t * 8, 8)])
        pltpu.sync_copy(x_hbm.at[pl.ds(r, 8), :], rows_buf.at[pl.ds(t * 8, 8), :])
        return carry

    lax.fori_loop(0, tail8, tail, 0)
    pltpu.sync_copy(rows_buf, acc.at[ids_buf], add=True)

    plsc.subcore_barrier()

    rpt = S // NS
    pltpu.sync_copy(acc.at[pl.ds(s * rpt, rpt)],
                    out_hbm.at[c, pl.ds(s * rpt, rpt), :])


@jax.jit
def _sc_segsum(x, ids):
    mesh = plsc.VectorSubcoreMesh(core_axis_name="c", subcore_axis_name="s")
    f = pl.kernel(
        _sc_body,
        out_type=jax.ShapeDtypeStruct((NC, S, D), jnp.float32),
        mesh=mesh,
        scratch_types=[
            pltpu.VMEM_SHARED((ACC_ROWS, D), jnp.float32),
            pltpu.VMEM((ACC_ROWS // NS, D), jnp.float32),
            pltpu.VMEM((CH, D), jnp.float32),
            pltpu.VMEM((CH,), jnp.int32),
        ],
    )
    return f(x, ids)


def _merge_body(p_ref, o_ref):
    o_ref[...] = p_ref[0] + p_ref[1]


@jax.jit
def _merge(partials):
    return pl.pallas_call(
        _merge_body,
        out_shape=jax.ShapeDtypeStruct((S, D), jnp.float32),
    )(partials)


def kernel(x, batch_indices):
    ids = batch_indices.astype(jnp.int32)
    partials = _sc_segsum(x, ids)
    pooled = _merge(partials)
    return (pooled, None)


# speedup vs baseline: 5.2187x; 1.2968x over previous
import functools

import jax
import jax.numpy as jnp
from jax import lax
from jax.experimental import pallas as pl
from jax.experimental.pallas import tpu as pltpu
from jax.experimental.pallas import tpu_sc as plsc

N = 100000
D = 128
S = 1024
NC = 2
NS = 16
NW = NC * NS
BASE = 3128
CH = 128
ACC_ROWS = 1040
DUMMY = 1024


def _sc_body(x_hbm, ids_hbm, out_hbm, acc, zbuf, rows0, rows1, ids0, ids1,
             sem0, sem1, sem0s, sem1s):
    c = lax.axis_index("c")
    s = lax.axis_index("s")
    wid = c * NS + s
    row0 = wid * BASE
    rows_w = jnp.minimum(BASE, N - row0)
    full = rows_w // CH
    tail8 = (rows_w - full * CH) // 8

    zeros16 = jnp.zeros((16,), jnp.float32)

    def zrow(i, carry):
        for k in range(D // 16):
            zbuf[i, pl.ds(k * 16, 16)] = zeros16
        return carry

    lax.fori_loop(0, ACC_ROWS // NS, zrow, 0)
    pltpu.sync_copy(zbuf, acc.at[pl.ds(s * (ACC_ROWS // NS), ACC_ROWS // NS)])

    @pl.when(full > 0)
    def _():
        pltpu.async_copy(ids_hbm.at[pl.ds(row0, CH)], ids0, sem0)
        pltpu.async_copy(x_hbm.at[pl.ds(row0, CH), :], rows0, sem0)

    dummy16 = jnp.full((16,), DUMMY, jnp.int32)
    for k in range(CH // 16):
        ids1[pl.ds(k * 16, 16)] = dummy16
    t0 = row0 + full * CH

    def tailb(t, carry):
        r = t0 + t * 8
        pltpu.sync_copy(ids_hbm.at[pl.ds(r, 8)], ids1.at[pl.ds(t * 8, 8)])
        pltpu.sync_copy(x_hbm.at[pl.ds(r, 8), :], rows1.at[pl.ds(t * 8, 8), :])
        return carry

    lax.fori_loop(0, tail8, tailb, 0)

    plsc.subcore_barrier()

    @pl.when(tail8 > 0)
    def _():
        pltpu.sync_copy(rows1, acc.at[ids1], add=True)

    def pair(g, carry):
        j0 = 2 * g
        r0 = row0 + j0 * CH
        pltpu.make_async_copy(ids_hbm.at[pl.ds(r0, CH)], ids0, sem0).wait()
        pltpu.make_async_copy(x_hbm.at[pl.ds(r0, CH), :], rows0, sem0).wait()
        pltpu.async_copy(rows0, acc.at[ids0], sem0s, add=True)

        @pl.when(j0 >= 1)
        def _():
            pltpu.make_async_copy(rows1, acc.at[ids1], sem1s).wait()

        @pl.when(j0 + 1 < full)
        def _():
            r1 = row0 + (j0 + 1) * CH
            pltpu.async_copy(ids_hbm.at[pl.ds(r1, CH)], ids1, sem1)
            pltpu.async_copy(x_hbm.at[pl.ds(r1, CH), :], rows1, sem1)

        @pl.when(j0 + 1 < full)
        def _():
            r1 = row0 + (j0 + 1) * CH
            pltpu.make_async_copy(ids_hbm.at[pl.ds(r1, CH)], ids1, sem1).wait()
            pltpu.make_async_copy(x_hbm.at[pl.ds(r1, CH), :], rows1, sem1).wait()
            pltpu.async_copy(rows1, acc.at[ids1], sem1s, add=True)
            pltpu.make_async_copy(rows0, acc.at[ids0], sem0s).wait()

            @pl.when(j0 + 2 < full)
            def _():
                r2 = row0 + (j0 + 2) * CH
                pltpu.async_copy(ids_hbm.at[pl.ds(r2, CH)], ids0, sem0)
                pltpu.async_copy(x_hbm.at[pl.ds(r2, CH), :], rows0, sem0)

        return carry

    lax.fori_loop(0, (full + 1) // 2, pair, 0)

    @pl.when((full > 0) & (full % 2 == 0))
    def _():
        pltpu.make_async_copy(rows1, acc.at[ids1], sem1s).wait()

    @pl.when(full % 2 == 1)
    def _():
        pltpu.make_async_copy(rows0, acc.at[ids0], sem0s).wait()

    plsc.subcore_barrier()

    rpt = S // NS
    pltpu.sync_copy(acc.at[pl.ds(s * rpt, rpt)],
                    out_hbm.at[c, pl.ds(s * rpt, rpt), :])


@jax.jit
def _sc_segsum(x, ids):
    mesh = plsc.VectorSubcoreMesh(core_axis_name="c", subcore_axis_name="s")
    f = pl.kernel(
        _sc_body,
        out_type=jax.ShapeDtypeStruct((NC, S, D), jnp.float32),
        mesh=mesh,
        scratch_types=[
            pltpu.VMEM_SHARED((ACC_ROWS, D), jnp.float32),
            pltpu.VMEM((ACC_ROWS // NS, D), jnp.float32),
            pltpu.VMEM((CH, D), jnp.float32),
            pltpu.VMEM((CH, D), jnp.float32),
            pltpu.VMEM((CH,), jnp.int32),
            pltpu.VMEM((CH,), jnp.int32),
            pltpu.SemaphoreType.DMA,
            pltpu.SemaphoreType.DMA,
            pltpu.SemaphoreType.DMA,
            pltpu.SemaphoreType.DMA,
        ],
    )
    return f(x, ids)


def _merge_body(p_ref, o_ref):
    o_ref[...] = p_ref[0] + p_ref[1]


@jax.jit
def _merge(partials):
    return pl.pallas_call(
        _merge_body,
        out_shape=jax.ShapeDtypeStruct((S, D), jnp.float32),
    )(partials)


def kernel(x, batch_indices):
    ids = batch_indices.astype(jnp.int32)
    partials = _sc_segsum(x, ids)
    pooled = _merge(partials)
    return (pooled, None)
